# baseline (speedup 1.0000x reference)
import functools

import jax
import jax.numpy as jnp
from jax import lax
from jax.experimental import pallas as pl
from jax.experimental.pallas import tpu as pltpu
from jax.experimental.pallas import tpu_sc as plsc

_NC = 2
_NS = 16
_NW = _NC * _NS
_LANES = 16

_B = 1024
_CTX = 20
_D = 64

_B_PER_W = _B // _NW
_IDX_PER_W = _B_PER_W * _CTX
_IDX_CHUNK = 128
_N_CHUNKS = _IDX_PER_W // _IDX_CHUNK

_RUN_BITS = 14
_HALF_RUN = 1 << (_RUN_BITS - 1)


def _context_vectors_sc(x_flat, emb_packed):
    mesh = plsc.VectorSubcoreMesh(core_axis_name="c", subcore_axis_name="s")

    @functools.partial(
        pl.kernel,
        mesh=mesh,
        out_type=jax.ShapeDtypeStruct((_B, _D), jnp.float32),
        compiler_params=pltpu.CompilerParams(needs_layout_passes=False),
        scratch_types=[
            pltpu.VMEM((_IDX_PER_W,), jnp.int32),
            pltpu.VMEM((_IDX_PER_W,), jnp.int32),
            pltpu.VMEM((_IDX_PER_W,), jnp.int32),
            pltpu.VMEM((_IDX_PER_W, 2 * _D), jnp.float32),
            pltpu.VMEM((_B_PER_W, _D), jnp.float32),
            pltpu.SemaphoreType.DMA,
        ],
    )
    def sc_kernel(x_hbm, emb_hbm, cv_hbm, raw_v, idx_v, h_v, rows_v, cv_v, sem):
        wid = lax.axis_index("s") * _NC + lax.axis_index("c")
        idx_base = wid * _IDX_PER_W
        b_base = wid * _B_PER_W

        pltpu.sync_copy(x_hbm.at[pl.ds(idx_base, _IDX_PER_W)], raw_v)
        for j in range(_IDX_PER_W // _LANES):
            sl = pl.ds(j * _LANES, _LANES)
            raw = raw_v[sl]
            hi = raw >> _RUN_BITS
            idx_v[sl] = (hi << (_RUN_BITS - 1)) | (raw & (_HALF_RUN - 1))
            h_v[sl] = (raw >> (_RUN_BITS - 1)) & 1

        copies = []
        for j in range(_N_CHUNKS):
            sl = pl.ds(j * _IDX_CHUNK, _IDX_CHUNK)
            copies.append(
                pltpu.async_copy(emb_hbm.at[idx_v.at[sl]], rows_v.at[sl], sem)
            )

        def pool(b):
            row0 = b * _CTX
            accs = [jnp.zeros((_LANES,), jnp.float32) for _ in range(_D // _LANES)]
            for l in range(_CTX):
                slot = row0 + l
                splat = jnp.zeros((_LANES,), jnp.int32) + slot
                odd = plsc.load_gather(h_v, [splat]) > 0
                for c in range(_D // _LANES):
                    lo = rows_v[slot, pl.ds(c * _LANES, _LANES)]
                    hi = rows_v[slot, pl.ds(_D + c * _LANES, _LANES)]
                    accs[c] = accs[c] + jnp.where(odd, hi, lo)
            for c in range(_D // _LANES):
                cv_v[b, pl.ds(c * _LANES, _LANES)] = accs[c] * (1.0 / _CTX)

        group = _B_PER_W // 4
        copies[0].wait()
        copies[1].wait()
        plsc.parallel_loop(0 * group, 1 * group, unroll=2)(pool)
        copies[2].wait()
        plsc.parallel_loop(1 * group, 2 * group, unroll=2)(pool)
        copies[3].wait()
        plsc.parallel_loop(2 * group, 3 * group, unroll=2)(pool)
        copies[4].wait()
        plsc.parallel_loop(3 * group, 4 * group, unroll=2)(pool)

        pltpu.sync_copy(cv_v, cv_hbm.at[pl.ds(b_base, _B_PER_W)])

    return sc_kernel(x_flat, emb_packed)


def _pack_table_tc(emb_t, v):
    run = 2 * _HALF_RUN
    half = _HALF_RUN
    n_blk = pl.cdiv(v, run)
    vp = n_blk * half

    def pack_body(e_ref, out_ref):
        t = e_ref[...]
        out_ref[:, 0:_D] = t[:, 0:half].T
        out_ref[:, _D : 2 * _D] = t[:, half:run].T

    return pl.pallas_call(
        pack_body,
        grid=(n_blk,),
        in_specs=[pl.BlockSpec((_D, run), lambda j: (0, j))],
        out_specs=pl.BlockSpec((half, 2 * _D), lambda j: (j, 0)),
        out_shape=jax.ShapeDtypeStruct((vp, 2 * _D), jnp.float32),
    )(emb_t)


def _project_tc(cv, w_t):
    v = w_t.shape[1]
    vb = 6144

    def mm_body(cv_ref, w_ref, out_ref):
        out_ref[...] = lax.dot_general(
            w_ref[...], cv_ref[...],
            (((0,), (1,)), ((), ())),
            preferred_element_type=jnp.float32,
        )

    return pl.pallas_call(
        mm_body,
        grid=(pl.cdiv(v, vb),),
        in_specs=[
            pl.BlockSpec((_B, _D), lambda j: (0, 0)),
            pl.BlockSpec((_D, vb), lambda j: (0, j)),
        ],
        out_specs=pl.BlockSpec((vb, _B), lambda j: (j, 0)),
        out_shape=jax.ShapeDtypeStruct((v, _B), jnp.float32),
    )(cv, w_t)


def kernel(x, in_embedding, out_weight):
    x_flat = jnp.reshape(x.astype(jnp.int32), (-1,))
    v, _ = in_embedding.shape
    emb_packed = _pack_table_tc(in_embedding.T, v)
    cv = _context_vectors_sc(x_flat, emb_packed)
    return _project_tc(cv, out_weight.T).T


# speedup vs baseline: 1.0230x; 1.0230x over previous
import functools

import jax
import jax.numpy as jnp
from jax import lax
from jax.experimental import pallas as pl
from jax.experimental.pallas import tpu as pltpu
from jax.experimental.pallas import tpu_sc as plsc

_NC = 2
_NS = 16
_NW = _NC * _NS
_LANES = 16

_B = 1024
_CTX = 20
_D = 64

_B_PER_W = _B // _NW
_IDX_PER_W = _B_PER_W * _CTX
_IDX_CHUNK = 128
_N_CHUNKS = _IDX_PER_W // _IDX_CHUNK

_RUN_BITS = 14
_HALF_RUN = 1 << (_RUN_BITS - 1)


def _context_vectors_sc(x_flat, emb_packed):
    mesh = plsc.VectorSubcoreMesh(core_axis_name="c", subcore_axis_name="s")

    @functools.partial(
        pl.kernel,
        mesh=mesh,
        out_type=jax.ShapeDtypeStruct((_B, _D), jnp.float32),
        compiler_params=pltpu.CompilerParams(needs_layout_passes=False),
        scratch_types=[
            pltpu.VMEM((_IDX_PER_W,), jnp.int32),
            pltpu.VMEM((_IDX_PER_W,), jnp.int32),
            pltpu.VMEM((_IDX_PER_W,), jnp.int32),
            pltpu.VMEM((_IDX_PER_W, 2 * _D), jnp.float32),
            pltpu.VMEM((_B_PER_W, _D), jnp.float32),
            pltpu.SemaphoreType.DMA,
        ],
    )
    def sc_kernel(x_hbm, emb_hbm, cv_hbm, raw_v, idx_v, h_v, rows_v, cv_v, sem):
        wid = lax.axis_index("s") * _NC + lax.axis_index("c")
        idx_base = wid * _IDX_PER_W
        b_base = wid * _B_PER_W

        pltpu.sync_copy(x_hbm.at[pl.ds(idx_base, _IDX_PER_W)], raw_v)
        for j in range(_IDX_PER_W // _LANES):
            sl = pl.ds(j * _LANES, _LANES)
            raw = raw_v[sl]
            hi = raw >> _RUN_BITS
            idx_v[sl] = (hi << (_RUN_BITS - 1)) | (raw & (_HALF_RUN - 1))
            h_v[sl] = (raw >> (_RUN_BITS - 1)) & 1

        copies = []
        for j in range(_N_CHUNKS):
            sl = pl.ds(j * _IDX_CHUNK, _IDX_CHUNK)
            copies.append(
                pltpu.async_copy(emb_hbm.at[idx_v.at[sl]], rows_v.at[sl], sem)
            )

        def body(b, carry):
            row0 = b * _CTX
            accs = [jnp.zeros((_LANES,), jnp.float32) for _ in range(_D // _LANES)]
            for l in range(_CTX):
                slot = row0 + l
                splat = jnp.zeros((_LANES,), jnp.int32) + slot
                odd = plsc.load_gather(h_v, [splat]) > 0
                for c in range(_D // _LANES):
                    lo = rows_v[slot, pl.ds(c * _LANES, _LANES)]
                    hi = rows_v[slot, pl.ds(_D + c * _LANES, _LANES)]
                    accs[c] = accs[c] + jnp.where(odd, hi, lo)
            for c in range(_D // _LANES):
                cv_v[b, pl.ds(c * _LANES, _LANES)] = accs[c] * (1.0 / _CTX)
            return carry

        group = _B_PER_W // 4
        copies[0].wait()
        copies[1].wait()
        lax.fori_loop(0 * group, 1 * group, body, 0)
        copies[2].wait()
        lax.fori_loop(1 * group, 2 * group, body, 0)
        copies[3].wait()
        lax.fori_loop(2 * group, 3 * group, body, 0)
        copies[4].wait()
        lax.fori_loop(3 * group, 4 * group, body, 0)

        pltpu.sync_copy(cv_v, cv_hbm.at[pl.ds(b_base, _B_PER_W)])

    return sc_kernel(x_flat, emb_packed)


def _pack_table_tc(emb_t, v):
    run = 2 * _HALF_RUN
    half = _HALF_RUN
    n_blk = pl.cdiv(v, run)
    vp = n_blk * half

    def pack_body(e_ref, out_ref):
        t = e_ref[...]
        out_ref[:, 0:_D] = t[:, 0:half].T
        out_ref[:, _D : 2 * _D] = t[:, half:run].T

    return pl.pallas_call(
        pack_body,
        grid=(n_blk,),
        in_specs=[pl.BlockSpec((_D, run), lambda j: (0, j))],
        out_specs=pl.BlockSpec((half, 2 * _D), lambda j: (j, 0)),
        out_shape=jax.ShapeDtypeStruct((vp, 2 * _D), jnp.float32),
    )(emb_t)


def _project_tc(cv, w_t):
    v = w_t.shape[1]
    vb = 6144

    def mm_body(cv_ref, w_ref, out_ref):
        out_ref[...] = lax.dot_general(
            w_ref[...], cv_ref[...],
            (((0,), (1,)), ((), ())),
            preferred_element_type=jnp.float32,
        )

    return pl.pallas_call(
        mm_body,
        grid=(pl.cdiv(v, vb),),
        in_specs=[
            pl.BlockSpec((_B, _D), lambda j: (0, 0)),
            pl.BlockSpec((_D, vb), lambda j: (0, j)),
        ],
        out_specs=pl.BlockSpec((vb, _B), lambda j: (j, 0)),
        out_shape=jax.ShapeDtypeStruct((v, _B), jnp.float32),
    )(cv, w_t)


def kernel(x, in_embedding, out_weight):
    x_flat = jnp.reshape(x.astype(jnp.int32), (-1,))
    v, _ = in_embedding.shape
    emb_packed = _pack_table_tc(in_embedding.T, v)
    cv = _context_vectors_sc(x_flat, emb_packed)
    return _project_tc(cv, out_weight.T).T
